# baseline (speedup 1.0000x reference)
import jax
import jax.numpy as jnp
from jax import lax
from jax.experimental import pallas as pl
from jax.experimental.pallas import tpu as pltpu
from jax.experimental.pallas import tpu_sc as plsc

N, E, D = 10000, 320000, 128

NC, NS, L = 2, 16, 16
NW = NC * NS
CH = 80
UNIT = NW * CH
BLOCK_UNITS = (13, 28, 28, 28, 28)
KB = len(BLOCK_UNITS)
NPAD = 10112
ROWS_PER_SUB = NPAD // NS


def _mish(t):
    u = 1.0 + jnp.exp(t)
    return t * (1.0 - 2.0 / (u * u + 1.0))


def _mm_mish_body(a_ref, w_ref, b_ref, o_ref):
    a = a_ref[...]
    w = w_ref[...]
    acc = lax.dot_general(a, w, (((1,), (1,)), ((), ())),
                          preferred_element_type=jnp.float32)
    o_ref[...] = _mish(acc + b_ref[...])


def _mm_mish(a, w, b, block_rows, out_rows=None, row_off=0):
    rows = a.shape[0]
    out_rows = rows if out_rows is None else out_rows
    grid = out_rows // block_rows
    off = row_off // block_rows
    return pl.pallas_call(
        _mm_mish_body,
        grid=(grid,),
        in_specs=[
            pl.BlockSpec((block_rows, D), lambda i: (off + i, 0)),
            pl.BlockSpec((D, D), lambda i: (0, 0)),
            pl.BlockSpec((1, D), lambda i: (0, 0)),
        ],
        out_specs=pl.BlockSpec((block_rows, D), lambda i: (i, 0)),
        out_shape=jax.ShapeDtypeStruct((out_rows, D), jnp.float32),
    )(a, w, b)


def _final_body(*refs):
    x_ref = refs[0]
    agg_refs = refs[1:-3]
    w_ref, b_ref, o_ref = refs[-3:]
    s = x_ref[...]
    for a in agg_refs:
        s = s + a[0]
    acc = lax.dot_general(s, w_ref[...], (((1,), (1,)), ((), ())),
                          preferred_element_type=jnp.float32)
    o_ref[...] = _mish(acc + b_ref[...])


def _final(x, aggs, w, b, block_rows=1000):
    grid = N // block_rows
    blk = pl.BlockSpec((block_rows, D), lambda i: (i, 0))
    agg_specs = []
    agg_args = []
    for a in aggs:
        for c in range(NC):
            agg_specs.append(
                pl.BlockSpec((1, block_rows, D), lambda i, c=c: (c, i, 0)))
            agg_args.append(a)
    return pl.pallas_call(
        _final_body,
        grid=(grid,),
        in_specs=[blk] + agg_specs + [
            pl.BlockSpec((D, D), lambda i: (0, 0)),
            pl.BlockSpec((1, D), lambda i: (0, 0)),
        ],
        out_specs=blk,
        out_shape=jax.ShapeDtypeStruct((N, D), jnp.float32),
    )(x, *agg_args, w, b)


def _sc_body(per_w, coff,
             x_hbm, y_hbm, src_hbm, dst2_hbm, out_hbm,
             src_all, dst_all, gx0, gx1, gy0, gy1, acc,
             sem_i, sem_g0, sem_g1, sem_s0, sem_s1):
    cid = lax.axis_index("c")
    sid = lax.axis_index("s")
    wid = sid * NC + cid

    gx = (gx0, gx1)
    gy = (gy0, gy1)
    sem_g = (sem_g0, sem_g1)
    sem_s = (sem_s0, sem_s1)

    lc0 = wid * per_w
    c0 = coff + lc0

    pltpu.async_copy(src_hbm.at[pl.ds(c0 * CH, per_w * CH)], src_all, sem_i)
    pltpu.async_copy(dst2_hbm.at[pl.ds(c0, per_w)], dst_all, sem_i)
    pltpu.make_async_copy(src_hbm.at[pl.ds(0, per_w * CH)], src_all,
                          sem_i).wait()
    pltpu.make_async_copy(dst2_hbm.at[pl.ds(0, per_w)], dst_all,
                          sem_i).wait()

    def issue_gather(i, b):
        base = (lc0 + i) * CH * D
        pltpu.async_copy(x_hbm.at[src_all.at[pl.ds(i * CH, CH)]], gx[b],
                         sem_g[b])
        pltpu.async_copy(y_hbm.at[pl.ds(base, CH * D)], gy[b], sem_g[b])

    def wait_gather(b):
        pltpu.make_async_copy(x_hbm.at[src_all.at[pl.ds(0, CH)]], gx[b],
                              sem_g[b]).wait()
        pltpu.make_async_copy(y_hbm.at[pl.ds(0, CH * D)], gy[b],
                              sem_g[b]).wait()

    def compute(b):
        gxb = gx[b]
        gyb = gy[b]

        def _row(e, _):
            for t in range(D // L):
                u = gyb[pl.ds(e * D + t * L, L)]
                a = gxb[e, pl.ds(t * L, L)]
                gxb[e, pl.ds(t * L, L)] = jnp.maximum(a + u, 0.0)
            return 0
        lax.fori_loop(0, CH, _row, 0)

    issue_gather(0, 0)

    def _z(t, _):
        e = t >> 3
        j = (t & 7) * L
        gx1[e, pl.ds(j, L)] = jnp.zeros((L,), jnp.float32)
        return 0
    lax.fori_loop(0, CH * D // L, _z, 0)
    base_rows = sid * ROWS_PER_SUB
    for kk in range(ROWS_PER_SUB // CH):
        pltpu.async_copy(gx1, acc.at[pl.ds(base_rows + kk * CH, CH)], sem_i)
    rem = ROWS_PER_SUB % CH
    if rem:
        pltpu.async_copy(gx1.at[pl.ds(0, rem)],
                         acc.at[pl.ds(base_rows + (ROWS_PER_SUB // CH) * CH,
                                      rem)], sem_i)
    for kk in range(ROWS_PER_SUB // CH):
        pltpu.make_async_copy(gx1, acc.at[pl.ds(base_rows, CH)],
                              sem_i).wait()
    if rem:
        pltpu.make_async_copy(gx1.at[pl.ds(0, rem)],
                              acc.at[pl.ds(base_rows, rem)], sem_i).wait()
    plsc.subcore_barrier()

    def _pair(t, _):
        for b in (0, 1):
            i = 2 * t + b

            @pl.when((i >= 1) & (i + 1 < per_w))
            def _():
                pltpu.make_async_copy(gx[1 - b], acc.at[dst_all.at[0, 0]],
                                      sem_s[1 - b]).wait()

            @pl.when(i + 1 < per_w)
            def _():
                issue_gather(i + 1, 1 - b)

            @pl.when(i < per_w)
            def _():
                wait_gather(b)
                compute(b)
                pltpu.async_copy(gx[b], acc.at[dst_all.at[i, 0]], sem_s[b],
                                 add=True)
        return 0
    lax.fori_loop(0, (per_w + 2) // 2, _pair, 0)
    pltpu.make_async_copy(gx[(per_w - 2) % 2], acc.at[dst_all.at[0, 0]],
                          sem_s[(per_w - 2) % 2]).wait()
    pltpu.make_async_copy(gx[(per_w - 1) % 2], acc.at[dst_all.at[0, 0]],
                          sem_s[(per_w - 1) % 2]).wait()
    plsc.subcore_barrier()

    pltpu.sync_copy(acc.at[pl.ds(base_rows, ROWS_PER_SUB)],
                    out_hbm.at[cid, pl.ds(base_rows, ROWS_PER_SUB)])


def _sc_agg(x, y, src, dst, per_w, coff):
    import functools as _ft
    mesh = plsc.VectorSubcoreMesh(core_axis_name="c", subcore_axis_name="s")
    f = pl.kernel(
        _ft.partial(_sc_body, per_w, coff),
        out_type=jax.ShapeDtypeStruct((NC, NPAD, D), jnp.float32),
        mesh=mesh,
        scratch_types=[
            pltpu.VMEM((per_w * CH,), jnp.int32),
            pltpu.VMEM((per_w, 1, CH), jnp.int32),
            pltpu.VMEM((CH, D), jnp.float32),
            pltpu.VMEM((CH, D), jnp.float32),
            pltpu.VMEM((CH * D,), jnp.float32),
            pltpu.VMEM((CH * D,), jnp.float32),
            pltpu.VMEM_SHARED((NPAD, D), jnp.float32),
            pltpu.SemaphoreType.DMA,
            pltpu.SemaphoreType.DMA,
            pltpu.SemaphoreType.DMA,
            pltpu.SemaphoreType.DMA,
            pltpu.SemaphoreType.DMA,
        ],
    )
    return f(x, y, src, dst)


def kernel(node_features, edge_features, targets, edge_index,
           W_dense, b_dense, W_edge, b_edge, W_out, b_out):
    del targets
    x = _mm_mish(node_features, W_dense, b_dense.reshape(1, D), 1000)
    b_e = b_edge.reshape(1, D)
    src_full = edge_index[0]
    dst_full = edge_index[1].reshape(E // CH, 1, CH)
    aggs = []
    off = 0
    for k in range(KB):
        eb = BLOCK_UNITS[k] * UNIT
        yk = _mm_mish(edge_features, W_edge, b_e, 2560,
                      out_rows=eb, row_off=off)
        ak = _sc_agg(x, yk.reshape(eb * D), src_full, dst_full,
                     BLOCK_UNITS[k], off // CH)
        aggs.append(ak)
        off += eb
    return _final(x, aggs, W_out, b_out.reshape(1, D))


# speedup vs baseline: 1.0495x; 1.0495x over previous
import jax
import jax.numpy as jnp
from jax import lax
from jax.experimental import pallas as pl
from jax.experimental.pallas import tpu as pltpu
from jax.experimental.pallas import tpu_sc as plsc

N, E, D = 10000, 320000, 128

NC, NS, L = 2, 16, 16
NW = NC * NS
CH = 80
UNIT = NW * CH
BLOCK_UNITS = (13, 37, 37, 38)
KB = len(BLOCK_UNITS)
NPAD = 10112
ROWS_PER_SUB = NPAD // NS


def _mish(t):
    u = 1.0 + jnp.exp(t)
    return t * (1.0 - 2.0 / (u * u + 1.0))


def _mm_mish_body(a_ref, w_ref, b_ref, o_ref):
    a = a_ref[...]
    w = w_ref[...]
    acc = lax.dot_general(a, w, (((1,), (1,)), ((), ())),
                          preferred_element_type=jnp.float32)
    o_ref[...] = _mish(acc + b_ref[...])


def _mm_mish(a, w, b, block_rows, out_rows=None, row_off=0):
    rows = a.shape[0]
    out_rows = rows if out_rows is None else out_rows
    grid = out_rows // block_rows
    off = row_off // block_rows
    return pl.pallas_call(
        _mm_mish_body,
        grid=(grid,),
        in_specs=[
            pl.BlockSpec((block_rows, D), lambda i: (off + i, 0)),
            pl.BlockSpec((D, D), lambda i: (0, 0)),
            pl.BlockSpec((1, D), lambda i: (0, 0)),
        ],
        out_specs=pl.BlockSpec((block_rows, D), lambda i: (i, 0)),
        out_shape=jax.ShapeDtypeStruct((out_rows, D), jnp.float32),
    )(a, w, b)


def _final_body(*refs):
    x_ref = refs[0]
    agg_refs = refs[1:-3]
    w_ref, b_ref, o_ref = refs[-3:]
    s = x_ref[...]
    for a in agg_refs:
        s = s + a[0]
    acc = lax.dot_general(s, w_ref[...], (((1,), (1,)), ((), ())),
                          preferred_element_type=jnp.float32)
    o_ref[...] = _mish(acc + b_ref[...])


def _final(x, aggs, w, b, block_rows=1000):
    grid = N // block_rows
    blk = pl.BlockSpec((block_rows, D), lambda i: (i, 0))
    agg_specs = []
    agg_args = []
    for a in aggs:
        for c in range(NC):
            agg_specs.append(
                pl.BlockSpec((1, block_rows, D), lambda i, c=c: (c, i, 0)))
            agg_args.append(a)
    return pl.pallas_call(
        _final_body,
        grid=(grid,),
        in_specs=[blk] + agg_specs + [
            pl.BlockSpec((D, D), lambda i: (0, 0)),
            pl.BlockSpec((1, D), lambda i: (0, 0)),
        ],
        out_specs=blk,
        out_shape=jax.ShapeDtypeStruct((N, D), jnp.float32),
    )(x, *agg_args, w, b)


def _sc_body(per_w, coff,
             x_hbm, y_hbm, ei_hbm, out_hbm,
             src_all, dst_all, gx0, gx1, gy0, gy1, acc,
             sem_i, sem_g0, sem_g1, sem_s0, sem_s1):
    cid = lax.axis_index("c")
    sid = lax.axis_index("s")
    wid = sid * NC + cid

    gx = (gx0, gx1)
    gy = (gy0, gy1)
    sem_g = (sem_g0, sem_g1)
    sem_s = (sem_s0, sem_s1)

    lc0 = wid * per_w
    c0 = coff + lc0

    pltpu.async_copy(ei_hbm.at[pl.ds(c0 * CH, per_w * CH)], src_all, sem_i)
    for kk in range(per_w):
        pltpu.async_copy(ei_hbm.at[pl.ds(E + (c0 + kk) * CH, CH)],
                         dst_all.at[kk, 0], sem_i)
    pltpu.make_async_copy(ei_hbm.at[pl.ds(0, per_w * CH)], src_all,
                          sem_i).wait()
    for kk in range(per_w):
        pltpu.make_async_copy(ei_hbm.at[pl.ds(0, CH)], dst_all.at[kk, 0],
                              sem_i).wait()

    def issue_gather(i, b):
        base = (lc0 + i) * CH * D
        pltpu.async_copy(x_hbm.at[src_all.at[pl.ds(i * CH, CH)]], gx[b],
                         sem_g[b])
        pltpu.async_copy(y_hbm.at[pl.ds(base, CH * D)], gy[b], sem_g[b])

    def wait_gather(b):
        pltpu.make_async_copy(x_hbm.at[src_all.at[pl.ds(0, CH)]], gx[b],
                              sem_g[b]).wait()
        pltpu.make_async_copy(y_hbm.at[pl.ds(0, CH * D)], gy[b],
                              sem_g[b]).wait()

    def compute(b):
        gxb = gx[b]
        gyb = gy[b]

        def _row(e, _):
            for t in range(D // L):
                u = gyb[pl.ds(e * D + t * L, L)]
                a = gxb[e, pl.ds(t * L, L)]
                gxb[e, pl.ds(t * L, L)] = jnp.maximum(a + u, 0.0)
            return 0
        lax.fori_loop(0, CH, _row, 0)

    issue_gather(0, 0)

    def _z(t, _):
        e = t >> 3
        j = (t & 7) * L
        gx1[e, pl.ds(j, L)] = jnp.zeros((L,), jnp.float32)
        return 0
    lax.fori_loop(0, CH * D // L, _z, 0)
    base_rows = sid * ROWS_PER_SUB
    for kk in range(ROWS_PER_SUB // CH):
        pltpu.async_copy(gx1, acc.at[pl.ds(base_rows + kk * CH, CH)], sem_i)
    rem = ROWS_PER_SUB % CH
    if rem:
        pltpu.async_copy(gx1.at[pl.ds(0, rem)],
                         acc.at[pl.ds(base_rows + (ROWS_PER_SUB // CH) * CH,
                                      rem)], sem_i)
    for kk in range(ROWS_PER_SUB // CH):
        pltpu.make_async_copy(gx1, acc.at[pl.ds(base_rows, CH)],
                              sem_i).wait()
    if rem:
        pltpu.make_async_copy(gx1.at[pl.ds(0, rem)],
                              acc.at[pl.ds(base_rows, rem)], sem_i).wait()
    plsc.subcore_barrier()

    def _pair(t, _):
        for b in (0, 1):
            i = 2 * t + b

            @pl.when((i >= 1) & (i + 1 < per_w))
            def _():
                pltpu.make_async_copy(gx[1 - b], acc.at[dst_all.at[0, 0]],
                                      sem_s[1 - b]).wait()

            @pl.when(i + 1 < per_w)
            def _():
                issue_gather(i + 1, 1 - b)

            @pl.when(i < per_w)
            def _():
                wait_gather(b)
                compute(b)
                pltpu.async_copy(gx[b], acc.at[dst_all.at[i, 0]], sem_s[b],
                                 add=True)
        return 0
    lax.fori_loop(0, (per_w + 2) // 2, _pair, 0)
    pltpu.make_async_copy(gx[(per_w - 2) % 2], acc.at[dst_all.at[0, 0]],
                          sem_s[(per_w - 2) % 2]).wait()
    pltpu.make_async_copy(gx[(per_w - 1) % 2], acc.at[dst_all.at[0, 0]],
                          sem_s[(per_w - 1) % 2]).wait()
    plsc.subcore_barrier()

    pltpu.sync_copy(acc.at[pl.ds(base_rows, ROWS_PER_SUB)],
                    out_hbm.at[cid, pl.ds(base_rows, ROWS_PER_SUB)])


def _sc_agg(x, y, ei_flat, per_w, coff):
    import functools as _ft
    mesh = plsc.VectorSubcoreMesh(core_axis_name="c", subcore_axis_name="s")
    f = pl.kernel(
        _ft.partial(_sc_body, per_w, coff),
        out_type=jax.ShapeDtypeStruct((NC, NPAD, D), jnp.float32),
        mesh=mesh,
        scratch_types=[
            pltpu.VMEM((per_w * CH,), jnp.int32),
            pltpu.VMEM((per_w, 1, CH), jnp.int32),
            pltpu.VMEM((CH, D), jnp.float32),
            pltpu.VMEM((CH, D), jnp.float32),
            pltpu.VMEM((CH * D,), jnp.float32),
            pltpu.VMEM((CH * D,), jnp.float32),
            pltpu.VMEM_SHARED((NPAD, D), jnp.float32),
            pltpu.SemaphoreType.DMA,
            pltpu.SemaphoreType.DMA,
            pltpu.SemaphoreType.DMA,
            pltpu.SemaphoreType.DMA,
            pltpu.SemaphoreType.DMA,
        ],
    )
    return f(x, y, ei_flat)


def kernel(node_features, edge_features, targets, edge_index,
           W_dense, b_dense, W_edge, b_edge, W_out, b_out):
    del targets
    x = _mm_mish(node_features, W_dense, b_dense.reshape(1, D), 1000)
    b_e = b_edge.reshape(1, D)
    ei_flat = edge_index.reshape(2 * E)
    aggs = []
    off = 0
    for k in range(KB):
        eb = BLOCK_UNITS[k] * UNIT
        yk = _mm_mish(edge_features, W_edge, b_e, 2560,
                      out_rows=eb, row_off=off)
        ak = _sc_agg(x, yk.reshape(eb * D), ei_flat,
                     BLOCK_UNITS[k], off // CH)
        aggs.append(ak)
        off += eb
    return _final(x, aggs, W_out, b_out.reshape(1, D))
